# Initial kernel scaffold; baseline (speedup 1.0000x reference)
import jax
import jax.numpy as jnp
from jax.experimental import pallas as pl


def kernel(x):
    raise NotImplementedError("write your pallas kernel here")


# speedup vs baseline: 4.6651x; 4.6651x over previous
import jax
import jax.numpy as jnp
from jax.experimental import pallas as pl

H, M = 16, 2048
BLK_R = 256
N_BLK = M // BLK_R
NEG_INF = float("-inf")


def _body(x_ref, diag_ref, off_ref):
    b = pl.program_id(1)
    blk = x_ref[0]
    row = jax.lax.broadcasted_iota(jnp.int32, (BLK_R, M), 0) + b * BLK_R
    col = jax.lax.broadcasted_iota(jnp.int32, (BLK_R, M), 1)
    on_diag = row == col
    off_m = jnp.max(jnp.where(on_diag, NEG_INF, blk))
    dia_m = jnp.max(jnp.where(on_diag, blk, NEG_INF))

    @pl.when(b == 0)
    def _():
        diag_ref[...] = jnp.full((1, 1, 128), NEG_INF, jnp.float32)
        off_ref[...] = jnp.full((1, 1, 128), NEG_INF, jnp.float32)

    diag_ref[...] = jnp.maximum(diag_ref[...], dia_m)
    off_ref[...] = jnp.maximum(off_ref[...], off_m)


def kernel(x):
    xs = x.reshape(H, M, M)
    diag, off = pl.pallas_call(
        _body,
        grid=(H, N_BLK),
        in_specs=[pl.BlockSpec((1, BLK_R, M), lambda h, b: (h, b, 0))],
        out_specs=[
            pl.BlockSpec((1, 1, 128), lambda h, b: (h, 0, 0)),
            pl.BlockSpec((1, 1, 128), lambda h, b: (h, 0, 0)),
        ],
        out_shape=[
            jax.ShapeDtypeStruct((H, 1, 128), jnp.float32),
            jax.ShapeDtypeStruct((H, 1, 128), jnp.float32),
        ],
    )(xs)
    return jnp.concatenate([diag[:, 0, 0], off[:, 0, 0]])[None, :]


# speedup vs baseline: 5.4964x; 1.1782x over previous
import jax
import jax.numpy as jnp
from jax.experimental import pallas as pl

H, M = 16, 2048
BLK_R = 256
N_BLK = M // BLK_R
NEG_INF = float("-inf")


def _body(x_ref, diag_ref, off_ref):
    b = pl.program_id(1)
    blk = x_ref[0]
    stripe = x_ref[0, :, pl.ds(b * BLK_R, BLK_R)]
    eye = (
        jax.lax.broadcasted_iota(jnp.int32, (BLK_R, BLK_R), 0)
        == jax.lax.broadcasted_iota(jnp.int32, (BLK_R, BLK_R), 1)
    )
    dia_m = jnp.max(jnp.where(eye, stripe, NEG_INF))
    stripe_off = jnp.max(jnp.where(eye, NEG_INF, stripe))
    colmax = jnp.max(blk, axis=0, keepdims=True)
    in_stripe = (
        jax.lax.broadcasted_iota(jnp.int32, (1, M), 1) // BLK_R
    ) == b
    off_m = jnp.maximum(jnp.max(jnp.where(in_stripe, NEG_INF, colmax)), stripe_off)

    @pl.when(b == 0)
    def _():
        diag_ref[...] = jnp.full((1, 1, 128), NEG_INF, jnp.float32)
        off_ref[...] = jnp.full((1, 1, 128), NEG_INF, jnp.float32)

    diag_ref[...] = jnp.maximum(diag_ref[...], dia_m)
    off_ref[...] = jnp.maximum(off_ref[...], off_m)


def kernel(x):
    xs = x.reshape(H, M, M)
    diag, off = pl.pallas_call(
        _body,
        grid=(H, N_BLK),
        in_specs=[pl.BlockSpec((1, BLK_R, M), lambda h, b: (h, b, 0))],
        out_specs=[
            pl.BlockSpec((1, 1, 128), lambda h, b: (h, 0, 0)),
            pl.BlockSpec((1, 1, 128), lambda h, b: (h, 0, 0)),
        ],
        out_shape=[
            jax.ShapeDtypeStruct((H, 1, 128), jnp.float32),
            jax.ShapeDtypeStruct((H, 1, 128), jnp.float32),
        ],
    )(xs)
    return jnp.concatenate([diag[:, 0, 0], off[:, 0, 0]])[None, :]


# speedup vs baseline: 7.5220x; 1.3685x over previous
import jax
import jax.numpy as jnp
from jax.experimental import pallas as pl

H, M = 16, 2048
BLK_R = 512
N_BLK = M // BLK_R
NEG_INF = float("-inf")


def _body(x_ref, diag_ref, off_ref):
    b = pl.program_id(1)
    blk = x_ref[0]
    stripe = x_ref[0, :, pl.ds(b * BLK_R, BLK_R)]
    eye = (
        jax.lax.broadcasted_iota(jnp.int32, (BLK_R, BLK_R), 0)
        == jax.lax.broadcasted_iota(jnp.int32, (BLK_R, BLK_R), 1)
    )
    dia_m = jnp.max(jnp.where(eye, stripe, NEG_INF))
    stripe_off = jnp.max(jnp.where(eye, NEG_INF, stripe))
    colmax = jnp.max(blk, axis=0, keepdims=True)
    in_stripe = (
        jax.lax.broadcasted_iota(jnp.int32, (1, M), 1) // BLK_R
    ) == b
    off_m = jnp.maximum(jnp.max(jnp.where(in_stripe, NEG_INF, colmax)), stripe_off)

    @pl.when(b == 0)
    def _():
        diag_ref[...] = jnp.full((1, 1, 128), NEG_INF, jnp.float32)
        off_ref[...] = jnp.full((1, 1, 128), NEG_INF, jnp.float32)

    diag_ref[...] = jnp.maximum(diag_ref[...], dia_m)
    off_ref[...] = jnp.maximum(off_ref[...], off_m)


def kernel(x):
    xs = x.reshape(H, M, M)
    diag, off = pl.pallas_call(
        _body,
        grid=(H, N_BLK),
        in_specs=[pl.BlockSpec((1, BLK_R, M), lambda h, b: (h, b, 0))],
        out_specs=[
            pl.BlockSpec((1, 1, 128), lambda h, b: (h, 0, 0)),
            pl.BlockSpec((1, 1, 128), lambda h, b: (h, 0, 0)),
        ],
        out_shape=[
            jax.ShapeDtypeStruct((H, 1, 128), jnp.float32),
            jax.ShapeDtypeStruct((H, 1, 128), jnp.float32),
        ],
    )(xs)
    return jnp.concatenate([diag[:, 0, 0], off[:, 0, 0]])[None, :]


# speedup vs baseline: 8.3240x; 1.1066x over previous
import jax
import jax.numpy as jnp
from jax.experimental import pallas as pl

H, M = 16, 2048
BLK_R = 1024
N_BLK = M // BLK_R
NEG_INF = float("-inf")


def _body(x_ref, diag_ref, off_ref):
    b = pl.program_id(1)
    blk = x_ref[0]
    stripe = x_ref[0, :, pl.ds(b * BLK_R, BLK_R)]
    eye = (
        jax.lax.broadcasted_iota(jnp.int32, (BLK_R, BLK_R), 0)
        == jax.lax.broadcasted_iota(jnp.int32, (BLK_R, BLK_R), 1)
    )
    dia_m = jnp.max(jnp.where(eye, stripe, NEG_INF))
    stripe_off = jnp.max(jnp.where(eye, NEG_INF, stripe))
    colmax = jnp.max(blk, axis=0, keepdims=True)
    in_stripe = (
        jax.lax.broadcasted_iota(jnp.int32, (1, M), 1) // BLK_R
    ) == b
    off_m = jnp.maximum(jnp.max(jnp.where(in_stripe, NEG_INF, colmax)), stripe_off)

    @pl.when(b == 0)
    def _():
        diag_ref[...] = jnp.full((1, 1, 128), NEG_INF, jnp.float32)
        off_ref[...] = jnp.full((1, 1, 128), NEG_INF, jnp.float32)

    diag_ref[...] = jnp.maximum(diag_ref[...], dia_m)
    off_ref[...] = jnp.maximum(off_ref[...], off_m)


def kernel(x):
    xs = x.reshape(H, M, M)
    diag, off = pl.pallas_call(
        _body,
        grid=(H, N_BLK),
        in_specs=[pl.BlockSpec((1, BLK_R, M), lambda h, b: (h, b, 0))],
        out_specs=[
            pl.BlockSpec((1, 1, 128), lambda h, b: (h, 0, 0)),
            pl.BlockSpec((1, 1, 128), lambda h, b: (h, 0, 0)),
        ],
        out_shape=[
            jax.ShapeDtypeStruct((H, 1, 128), jnp.float32),
            jax.ShapeDtypeStruct((H, 1, 128), jnp.float32),
        ],
    )(xs)
    return jnp.concatenate([diag[:, 0, 0], off[:, 0, 0]])[None, :]
